# Initial kernel scaffold; baseline (speedup 1.0000x reference)
import jax
import jax.numpy as jnp
from jax.experimental import pallas as pl


def kernel(hidden_states, edge_index, batch, W1, b1, W2, b2, W3, b3, rms_weight, Wc, bc):
    raise NotImplementedError("write your pallas kernel here")


# speedup vs baseline: 8.4292x; 8.4292x over previous
import functools

import jax
import jax.numpy as jnp
from jax import lax
from jax.experimental import pallas as pl
from jax.experimental.pallas import tpu as pltpu
from jax.experimental.pallas import tpu_sc as plsc

N = 10000
E = 320000
G = 16
D_IN = 128
H = 512
I_DIM = 2048
O_DIM = 2000
C_DIM = 2000
EPS = 1e-08

NPAD = 10240
FC = 128
NC = 2
NS = 16
NW = NC * NS
EPT = NPAD
BI = 128
NBLK = EPT // BI
SB = 8
NSB = NBLK // SB
RPT = NPAD // NS

_SC_MESH = plsc.VectorSubcoreMesh(
    core_axis_name="c", subcore_axis_name="s", num_cores=NC, num_subcores=NS)


def _make_deg():
  @functools.partial(
      pl.kernel,
      out_type=jax.ShapeDtypeStruct((NC * NPAD,), jnp.float32),
      mesh=_SC_MESH,
      scratch_types=[
          pltpu.VMEM((NBLK, BI), jnp.int32),
          pltpu.VMEM((RPT,), jnp.float32),
          pltpu.VMEM((BI,), jnp.float32),
          pltpu.VMEM_SHARED((NPAD,), jnp.float32),
          pltpu.SemaphoreType.DMA,
      ],
  )
  def deg_kernel(dst_hbm, out_hbm, didx, zbuf, ones_v, acc, sem):
    c = lax.axis_index("c")
    s = lax.axis_index("s")
    wid = c * NS + s
    pltpu.sync_copy(dst_hbm.at[wid], didx)
    zv = jnp.zeros((16,), jnp.float32)
    ov = jnp.ones((16,), jnp.float32)
    def fill(i, _):
      zbuf[pl.ds(i * 16, 16)] = zv
      return 0
    lax.fori_loop(0, RPT // 16, fill, 0)
    for i in range(BI // 16):
      ones_v[pl.ds(i * 16, 16)] = ov
    pltpu.sync_copy(zbuf, acc.at[pl.ds(s * RPT, RPT)])
    plsc.subcore_barrier()

    def round_body(r, _):
      for b in range(8):
        j = r * 8 + b
        pltpu.async_copy(ones_v, acc.at[didx.at[j]], sem, add=True)
      pltpu.make_async_copy(
          dst_hbm.at[0, pl.ds(0, 8)], didx.at[pl.ds(0, 8)], sem).wait()
      return 0
    lax.fori_loop(0, NBLK // 8, round_body, 0)
    plsc.subcore_barrier()
    pltpu.sync_copy(acc.at[pl.ds(s * RPT, RPT)],
                    out_hbm.at[pl.ds(c * NPAD + s * RPT, RPT)])

  return deg_kernel


def _make_spmm(ncs):
  @functools.partial(
      pl.kernel,
      out_type=jax.ShapeDtypeStruct((NC, ncs, NPAD, FC), jnp.float32),
      mesh=_SC_MESH,
      scratch_types=[
          pltpu.VMEM((2, SB, BI), jnp.int32),
          pltpu.VMEM((2, SB, BI), jnp.int32),
          pltpu.VMEM((2, BI, FC), jnp.float32),
          pltpu.VMEM_SHARED((NPAD, FC), jnp.float32),
          pltpu.SemaphoreType.DMA,
          pltpu.SemaphoreType.DMA,
          pltpu.SemaphoreType.DMA,
          pltpu.SemaphoreType.DMA,
          pltpu.SemaphoreType.DMA,
          pltpu.SemaphoreType.DMA,
      ],
  )
  def spmm_kernel(xn_hbm, src_hbm, dst_hbm, out_hbm,
                  sbuf, dbuf, rows, acc, g0, g1, s0, s1, i0, i1):
    gsem = (g0, g1)
    ssem = (s0, s1)
    isem = (i0, i1)
    c = lax.axis_index("c")
    s = lax.axis_index("s")
    wid = c * NS + s
    zv = jnp.zeros((16,), jnp.float32)

    def gwait(b):
      pltpu.make_async_copy(
          xn_hbm.at[0, pl.ds(0, BI)], rows.at[b], gsem[b]).wait()

    def swait(b):
      pltpu.make_async_copy(
          xn_hbm.at[0, pl.ds(0, BI)], rows.at[b], ssem[b]).wait()

    def iwait(ib):
      pltpu.make_async_copy(
          src_hbm.at[0, pl.ds(0, SB)], sbuf.at[ib], isem[ib]).wait()
      pltpu.make_async_copy(
          src_hbm.at[0, pl.ds(0, SB)], dbuf.at[ib], isem[ib]).wait()

    def refill(ib, sb_next):
      pltpu.async_copy(src_hbm.at[wid, pl.ds(sb_next * SB, SB)],
                       sbuf.at[ib], isem[ib])
      pltpu.async_copy(dst_hbm.at[wid, pl.ds(sb_next * SB, SB)],
                       dbuf.at[ib], isem[ib])

    for ci in range(ncs):
      def fillz(i, _):
        for k in range(FC // 16):
          rows[0, i, pl.ds(k * 16, 16)] = zv
        return 0
      lax.fori_loop(0, BI, fillz, 0)
      for z in range(RPT // BI):
        pltpu.sync_copy(rows.at[0], acc.at[pl.ds(s * RPT + z * BI, BI)])
      plsc.subcore_barrier()

      pltpu.sync_copy(src_hbm.at[wid, pl.ds(0, SB)], sbuf.at[0])
      pltpu.sync_copy(dst_hbm.at[wid, pl.ds(0, SB)], dbuf.at[0])
      pltpu.async_copy(xn_hbm.at[ci].at[sbuf.at[0, 0]], rows.at[0], gsem[0])

      def pair(p, _):
        for sbp in range(2):
          ib = sbp
          io = 1 - sbp
          sb = p * 2 + sbp
          for jl in range(SB):
            b = jl % 2
            bo = 1 - b
            gwait(b)
            pltpu.async_copy(rows.at[b], acc.at[dbuf.at[ib, jl]], ssem[b],
                             add=True)
            if jl == 3:
              if sbp == 0:
                refill(io, sb + 1)
              else:
                @pl.when(p < NSB // 2 - 1)
                def _():
                  refill(io, sb + 1)
            if jl < SB - 1:
              if sbp == 0 and jl == 0:
                @pl.when(p > 0)
                def _():
                  swait(bo)
              else:
                swait(bo)
              pltpu.async_copy(xn_hbm.at[ci].at[sbuf.at[ib, jl + 1]],
                               rows.at[bo], gsem[bo])
            else:
              if sbp == 0:
                swait(bo)
                iwait(io)
                pltpu.async_copy(xn_hbm.at[ci].at[sbuf.at[io, 0]],
                                 rows.at[bo], gsem[bo])
              else:
                @pl.when(p < NSB // 2 - 1)
                def _():
                  swait(bo)
                  iwait(io)
                  pltpu.async_copy(xn_hbm.at[ci].at[sbuf.at[io, 0]],
                                   rows.at[bo], gsem[bo])
        return 0
      lax.fori_loop(0, NSB // 2, pair, 0)
      swait(0)
      swait(1)
      plsc.subcore_barrier()
      pltpu.sync_copy(acc.at[pl.ds(s * RPT, RPT)],
                      out_hbm.at[c, ci, pl.ds(s * RPT, RPT)])

  return spmm_kernel


_DEG = _make_deg()
_SPMM = {ncs: _make_spmm(ncs) for ncs in (1, 4, 16)}


def _xn0(x_p, degp):
  bm = 512
  def body(xr, dgr, xnr, dvr):
    d = sum(dgr[cc] for cc in range(NC)) + 1.0
    dv = lax.rsqrt(d)
    dvr[...] = dv
    xnr[0] = xr[...] * dv
  return pl.pallas_call(
      body,
      grid=(NPAD // bm,),
      in_specs=[
          pl.BlockSpec((bm, FC), lambda i: (i, 0)),
          pl.BlockSpec((NC, bm, 1), lambda i: (0, i, 0)),
      ],
      out_specs=(
          pl.BlockSpec((1, bm, FC), lambda i: (0, i, 0)),
          pl.BlockSpec((bm, 1), lambda i: (i, 0)),
      ),
      out_shape=(
          jax.ShapeDtypeStruct((1, NPAD, FC), jnp.float32),
          jax.ShapeDtypeStruct((NPAD, 1), jnp.float32),
      ),
  )(x_p, degp)


def _mm_silu(zp, xn, wr, b2, dinv, ncs_in, fout):
  bm = 512
  bn = 256
  ncs_out = fout // FC
  def body(zpr, xnr, wrr, brr, dvr, outr, accr):
    k = pl.program_id(2)
    @pl.when(k == 0)
    def _():
      accr[...] = jnp.zeros_like(accr)
    z = sum(zpr[cc, 0] for cc in range(NC)) + xnr[0]
    accr[...] += jnp.dot(z, wrr[0], preferred_element_type=jnp.float32)
    @pl.when(k == ncs_in - 1)
    def _():
      dv = dvr[...]
      y = accr[...] * dv + brr[...]
      h = y * jax.nn.sigmoid(y)
      o = h * dv
      for t in range(bn // FC):
        outr[t] = o[:, t * FC:(t + 1) * FC]
  return pl.pallas_call(
      body,
      grid=(NPAD // bm, fout // bn, ncs_in),
      in_specs=[
          pl.BlockSpec((NC, 1, bm, FC), lambda i, j, k: (0, k, i, 0)),
          pl.BlockSpec((1, bm, FC), lambda i, j, k: (k, i, 0)),
          pl.BlockSpec((1, FC, bn), lambda i, j, k: (k, 0, j)),
          pl.BlockSpec((1, bn), lambda i, j, k: (0, j)),
          pl.BlockSpec((bm, 1), lambda i, j, k: (i, 0)),
      ],
      out_specs=pl.BlockSpec((bn // FC, bm, FC), lambda i, j, k: (j, i, 0)),
      out_shape=jax.ShapeDtypeStruct((ncs_out, NPAD, FC), jnp.float32),
      scratch_shapes=[pltpu.VMEM((bm, bn), jnp.float32)],
      compiler_params=pltpu.CompilerParams(
          dimension_semantics=("parallel", "parallel", "arbitrary")),
  )(zp, xn, wr, b2, dinv)


def _mm_rms(zp, xn, wr, b2, dinv, rmsw):
  bm = 256
  ncs_in = I_DIM // FC
  fo = 2048
  def body(zpr, xnr, wrr, brr, dvr, rwr, outr, accr):
    k = pl.program_id(1)
    @pl.when(k == 0)
    def _():
      accr[...] = jnp.zeros_like(accr)
    z = sum(zpr[cc, 0] for cc in range(NC)) + xnr[0]
    accr[...] += jnp.dot(z, wrr[0], preferred_element_type=jnp.float32)
    @pl.when(k == ncs_in - 1)
    def _():
      y = accr[...] * dvr[...] + brr[...]
      var = jnp.sum(y * y, axis=1, keepdims=True) * (1.0 / O_DIM)
      outr[...] = y * lax.rsqrt(var + EPS) * rwr[...]
  return pl.pallas_call(
      body,
      grid=(NPAD // bm, ncs_in),
      in_specs=[
          pl.BlockSpec((NC, 1, bm, FC), lambda i, k: (0, k, i, 0)),
          pl.BlockSpec((1, bm, FC), lambda i, k: (k, i, 0)),
          pl.BlockSpec((1, FC, fo), lambda i, k: (k, 0, 0)),
          pl.BlockSpec((1, fo), lambda i, k: (0, 0)),
          pl.BlockSpec((bm, 1), lambda i, k: (i, 0)),
          pl.BlockSpec((1, fo), lambda i, k: (0, 0)),
      ],
      out_specs=pl.BlockSpec((bm, fo), lambda i, k: (i, 0)),
      out_shape=jax.ShapeDtypeStruct((NPAD, fo), jnp.float32),
      scratch_shapes=[pltpu.VMEM((bm, fo), jnp.float32)],
      compiler_params=pltpu.CompilerParams(
          dimension_semantics=("parallel", "arbitrary")),
  )(zp, xn, wr, b2, dinv, rmsw)


def _pool_cls(hf, batch2, wc, bc2):
  bm = 512
  fo = 2048
  def body(hfr, btr, wcr, bcr, outr, poolr, cntr):
    i = pl.program_id(0)
    ni = pl.num_programs(0)
    @pl.when(i == 0)
    def _():
      poolr[...] = jnp.zeros_like(poolr)
      cntr[...] = jnp.zeros_like(cntr)
    m = (btr[...] == lax.broadcasted_iota(jnp.int32, (bm, G), 1)
         ).astype(jnp.float32)
    poolr[...] += lax.dot_general(m, hfr[...], (((0,), (0,)), ((), ())),
                                  preferred_element_type=jnp.float32)
    cntr[...] += lax.dot_general(m, jnp.ones((bm, 1), jnp.float32),
                                 (((0,), (0,)), ((), ())),
                                 preferred_element_type=jnp.float32)
    @pl.when(i == ni - 1)
    def _():
      pooled = poolr[...] / jnp.maximum(cntr[...], 1.0)
      outr[...] = jnp.dot(pooled, wcr[...],
                          preferred_element_type=jnp.float32) + bcr[...]
  return pl.pallas_call(
      body,
      grid=(NPAD // bm,),
      in_specs=[
          pl.BlockSpec((bm, fo), lambda i: (i, 0)),
          pl.BlockSpec((bm, 1), lambda i: (i, 0)),
          pl.BlockSpec((fo, fo), lambda i: (0, 0)),
          pl.BlockSpec((1, fo), lambda i: (0, 0)),
      ],
      out_specs=pl.BlockSpec((G, fo), lambda i: (0, 0)),
      out_shape=jax.ShapeDtypeStruct((G, fo), jnp.float32),
      scratch_shapes=[pltpu.VMEM((G, fo), jnp.float32),
                      pltpu.VMEM((G, 1), jnp.float32)],
      compiler_params=pltpu.CompilerParams(
          dimension_semantics=("arbitrary",)),
  )(hf, batch2, wc, bc2)


def kernel(hidden_states, edge_index, batch, W1, b1, W2, b2, W3, b3,
           rms_weight, Wc, bc):
  f32 = jnp.float32
  x_p = jnp.pad(hidden_states.astype(f32), ((0, NPAD - N), (0, 0)))
  src = edge_index[0]
  dst = edge_index[1]
  pe = NW * EPT - E
  pad_src = jnp.arange(pe, dtype=jnp.int32) % N
  pad_dst = N + jnp.arange(pe, dtype=jnp.int32) % (NPAD - N)
  srcp = jnp.concatenate([src, pad_src]).reshape(NW, NBLK, BI)
  dstp = jnp.concatenate([dst, pad_dst]).reshape(NW, NBLK, BI)
  batch2 = jnp.pad(batch, (0, NPAD - N),
                   constant_values=G).reshape(NPAD, 1)

  W1r = W1.reshape(1, FC, H)
  W2r = W2.reshape(H // FC, FC, I_DIM)
  W3r = jnp.pad(W3, ((0, 0), (0, 2048 - O_DIM))).reshape(I_DIM // FC, FC, 2048)
  b1r = b1.reshape(1, H)
  b2r = b2.reshape(1, I_DIM)
  b3r = jnp.pad(b3, (0, 2048 - O_DIM)).reshape(1, 2048)
  rmswr = jnp.pad(rms_weight, (0, 2048 - O_DIM)).reshape(1, 2048)
  wcp = jnp.pad(Wc, ((0, 2048 - O_DIM), (0, 2048 - C_DIM)))
  bcp = jnp.pad(bc, (0, 2048 - C_DIM)).reshape(1, 2048)

  degp = _DEG(dstp).reshape(NC, NPAD, 1)
  xn0, dinv = _xn0(x_p, degp)
  z1 = _SPMM[1](xn0, srcp, dstp)
  xn1 = _mm_silu(z1, xn0, W1r, b1r, dinv, 1, H)
  z2 = _SPMM[4](xn1, srcp, dstp)
  xn2 = _mm_silu(z2, xn1, W2r, b2r, dinv, 4, I_DIM)
  z3 = _SPMM[16](xn2, srcp, dstp)
  hf = _mm_rms(z3, xn2, W3r, b3r, dinv, rmswr)
  outp = _pool_cls(hf, batch2, wcp, bcp)
  return outp[:, :C_DIM]


# speedup vs baseline: 9.0914x; 1.0786x over previous
import functools

import jax
import jax.numpy as jnp
from jax import lax
from jax.experimental import pallas as pl
from jax.experimental.pallas import tpu as pltpu
from jax.experimental.pallas import tpu_sc as plsc

N = 10000
E = 320000
G = 16
D_IN = 128
H = 512
I_DIM = 2048
O_DIM = 2000
C_DIM = 2000
EPS = 1e-08

NPAD = 10240
FC = 128
NC = 2
NS = 16
NW = NC * NS
EPT = NPAD
BI = 128
NBLK = EPT // BI
BG = 80
NBG = EPT // BG
SBB = 8
NSB = NBG // SBB
IDXW = BG * SBB
RPT = NPAD // NS

_SC_MESH = plsc.VectorSubcoreMesh(
    core_axis_name="c", subcore_axis_name="s", num_cores=NC, num_subcores=NS)


def _make_deg():
  @functools.partial(
      pl.kernel,
      out_type=jax.ShapeDtypeStruct((NC * NPAD,), jnp.float32),
      mesh=_SC_MESH,
      scratch_types=[
          pltpu.VMEM((NBLK, BI), jnp.int32),
          pltpu.VMEM((RPT,), jnp.float32),
          pltpu.VMEM((BI,), jnp.float32),
          pltpu.VMEM_SHARED((NPAD,), jnp.float32),
          pltpu.SemaphoreType.DMA,
      ],
  )
  def deg_kernel(dst_hbm, out_hbm, didx, zbuf, ones_v, acc, sem):
    c = lax.axis_index("c")
    s = lax.axis_index("s")
    wid = c * NS + s
    pltpu.sync_copy(dst_hbm.at[wid], didx)
    zv = jnp.zeros((16,), jnp.float32)
    ov = jnp.ones((16,), jnp.float32)
    def fill(i, _):
      zbuf[pl.ds(i * 16, 16)] = zv
      return 0
    lax.fori_loop(0, RPT // 16, fill, 0)
    for i in range(BI // 16):
      ones_v[pl.ds(i * 16, 16)] = ov
    pltpu.sync_copy(zbuf, acc.at[pl.ds(s * RPT, RPT)])
    plsc.subcore_barrier()

    def round_body(r, _):
      for b in range(8):
        j = r * 8 + b
        pltpu.async_copy(ones_v, acc.at[didx.at[j]], sem, add=True)
      pltpu.make_async_copy(
          dst_hbm.at[0, pl.ds(0, 8)], didx.at[pl.ds(0, 8)], sem).wait()
      return 0
    lax.fori_loop(0, NBLK // 8, round_body, 0)
    plsc.subcore_barrier()
    pltpu.sync_copy(acc.at[pl.ds(s * RPT, RPT)],
                    out_hbm.at[pl.ds(c * NPAD + s * RPT, RPT)])

  return deg_kernel


def _make_spmm(ncs):
  @functools.partial(
      pl.kernel,
      out_type=jax.ShapeDtypeStruct((NC, ncs, NPAD, FC), jnp.float32),
      mesh=_SC_MESH,
      scratch_types=[
          pltpu.VMEM((IDXW,), jnp.int32),
          pltpu.VMEM((IDXW,), jnp.int32),
          pltpu.VMEM((IDXW,), jnp.int32),
          pltpu.VMEM((IDXW,), jnp.int32),
          pltpu.VMEM((4, BG, FC), jnp.float32),
          pltpu.VMEM_SHARED((NPAD, FC), jnp.float32),
          [pltpu.SemaphoreType.DMA] * 4,
          [pltpu.SemaphoreType.DMA] * 4,
          [pltpu.SemaphoreType.DMA] * 2,
      ],
  )
  def spmm_kernel(xn_hbm, src_hbm, dst_hbm, out_hbm,
                  sbuf0, sbuf1, dbuf0, dbuf1, rows, acc, gsem, ssem, isem):
    sbufs = (sbuf0, sbuf1)
    dbufs = (dbuf0, dbuf1)
    c = lax.axis_index("c")
    s = lax.axis_index("s")
    wid = c * NS + s
    zv = jnp.zeros((16,), jnp.float32)

    def gwait(b):
      pltpu.make_async_copy(
          xn_hbm.at[0, pl.ds(0, BG)], rows.at[b], gsem[b]).wait()

    def swait(b):
      pltpu.make_async_copy(
          xn_hbm.at[0, pl.ds(0, BG)], rows.at[b], ssem[b]).wait()

    def iwait(ib):
      pltpu.make_async_copy(
          src_hbm.at[pl.ds(0, IDXW)], sbufs[ib], isem[ib]).wait()
      pltpu.make_async_copy(
          src_hbm.at[pl.ds(0, IDXW)], dbufs[ib], isem[ib]).wait()

    def refill(ib, sb_next):
      off = wid * EPT + sb_next * IDXW
      pltpu.async_copy(src_hbm.at[pl.ds(off, IDXW)], sbufs[ib], isem[ib])
      pltpu.async_copy(dst_hbm.at[pl.ds(off, IDXW)], dbufs[ib], isem[ib])

    def gissue(ci, ib, jl, b):
      pltpu.async_copy(xn_hbm.at[ci].at[sbufs[ib].at[pl.ds(jl * BG, BG)]],
                       rows.at[b], gsem[b])

    for ci in range(ncs):
      def fillz(i, _):
        for k in range(FC // 16):
          rows[0, i, pl.ds(k * 16, 16)] = zv
        return 0
      lax.fori_loop(0, BG, fillz, 0)
      for z in range(RPT // BG):
        pltpu.sync_copy(rows.at[0], acc.at[pl.ds(s * RPT + z * BG, BG)])
      plsc.subcore_barrier()

      pltpu.sync_copy(src_hbm.at[pl.ds(wid * EPT, IDXW)], sbuf0)
      pltpu.sync_copy(dst_hbm.at[pl.ds(wid * EPT, IDXW)], dbuf0)
      gissue(ci, 0, 0, 0)
      gissue(ci, 0, 1, 1)

      def pair(p, _):
        for sbp in range(2):
          ib = sbp
          io = 1 - sbp
          sb = p * 2 + sbp
          for jl in range(SBB):
            b = jl % 4
            br = (jl + 2) % 4
            gwait(b)
            pltpu.async_copy(
                rows.at[b], acc.at[dbufs[ib].at[pl.ds(jl * BG, BG)]],
                ssem[b], add=True)
            if jl == 3:
              if sbp == 0:
                refill(io, sb + 1)
              else:
                @pl.when(p < NSB // 2 - 1)
                def _():
                  refill(io, sb + 1)
            if sbp == 0 and jl < 2:
              @pl.when(p > 0)
              def _():
                swait(br)
              gissue(ci, ib, jl + 2, br)
            elif jl < SBB - 2:
              swait(br)
              gissue(ci, ib, jl + 2, br)
            else:
              if sbp == 0:
                swait(br)
                if jl == SBB - 2:
                  iwait(io)
                gissue(ci, io, jl - (SBB - 2), br)
              else:
                @pl.when(p < NSB // 2 - 1)
                def _():
                  swait(br)
                  if jl == SBB - 2:
                    iwait(io)
                  gissue(ci, io, jl - (SBB - 2), br)
        return 0
      lax.fori_loop(0, NSB // 2, pair, 0)
      for b in range(4):
        swait(b)
      plsc.subcore_barrier()
      pltpu.sync_copy(acc.at[pl.ds(s * RPT, RPT)],
                      out_hbm.at[c, ci, pl.ds(s * RPT, RPT)])

  return spmm_kernel


_DEG = _make_deg()
_SPMM = {ncs: _make_spmm(ncs) for ncs in (1, 4, 16)}


def _xn0(x_p, degp):
  bm = 512
  def body(xr, dgr, xnr, dvr):
    d = sum(dgr[cc] for cc in range(NC)) + 1.0
    dv = lax.rsqrt(d)
    dvr[...] = dv
    xnr[0] = xr[...] * dv
  return pl.pallas_call(
      body,
      grid=(NPAD // bm,),
      in_specs=[
          pl.BlockSpec((bm, FC), lambda i: (i, 0)),
          pl.BlockSpec((NC, bm, 1), lambda i: (0, i, 0)),
      ],
      out_specs=(
          pl.BlockSpec((1, bm, FC), lambda i: (0, i, 0)),
          pl.BlockSpec((bm, 1), lambda i: (i, 0)),
      ),
      out_shape=(
          jax.ShapeDtypeStruct((1, NPAD, FC), jnp.float32),
          jax.ShapeDtypeStruct((NPAD, 1), jnp.float32),
      ),
  )(x_p, degp)


def _mm_silu(zp, xn, wr, b2, dinv, ncs_in, fout):
  bm = 512
  bn = 256
  ncs_out = fout // FC
  def body(zpr, xnr, wrr, brr, dvr, outr, accr):
    k = pl.program_id(2)
    @pl.when(k == 0)
    def _():
      accr[...] = jnp.zeros_like(accr)
    z = sum(zpr[cc, 0] for cc in range(NC)) + xnr[0]
    accr[...] += jnp.dot(z, wrr[0], preferred_element_type=jnp.float32)
    @pl.when(k == ncs_in - 1)
    def _():
      dv = dvr[...]
      y = accr[...] * dv + brr[...]
      h = y * jax.nn.sigmoid(y)
      o = h * dv
      for t in range(bn // FC):
        outr[t] = o[:, t * FC:(t + 1) * FC]
  return pl.pallas_call(
      body,
      grid=(NPAD // bm, fout // bn, ncs_in),
      in_specs=[
          pl.BlockSpec((NC, 1, bm, FC), lambda i, j, k: (0, k, i, 0)),
          pl.BlockSpec((1, bm, FC), lambda i, j, k: (k, i, 0)),
          pl.BlockSpec((1, FC, bn), lambda i, j, k: (k, 0, j)),
          pl.BlockSpec((1, bn), lambda i, j, k: (0, j)),
          pl.BlockSpec((bm, 1), lambda i, j, k: (i, 0)),
      ],
      out_specs=pl.BlockSpec((bn // FC, bm, FC), lambda i, j, k: (j, i, 0)),
      out_shape=jax.ShapeDtypeStruct((ncs_out, NPAD, FC), jnp.float32),
      scratch_shapes=[pltpu.VMEM((bm, bn), jnp.float32)],
      compiler_params=pltpu.CompilerParams(
          dimension_semantics=("parallel", "parallel", "arbitrary")),
  )(zp, xn, wr, b2, dinv)


def _mm_rms(zp, xn, wr, b2, dinv, rmsw):
  bm = 256
  ncs_in = I_DIM // FC
  fo = 2048
  def body(zpr, xnr, wrr, brr, dvr, rwr, outr, accr):
    k = pl.program_id(1)
    @pl.when(k == 0)
    def _():
      accr[...] = jnp.zeros_like(accr)
    z = sum(zpr[cc, 0] for cc in range(NC)) + xnr[0]
    accr[...] += jnp.dot(z, wrr[0], preferred_element_type=jnp.float32)
    @pl.when(k == ncs_in - 1)
    def _():
      y = accr[...] * dvr[...] + brr[...]
      var = jnp.sum(y * y, axis=1, keepdims=True) * (1.0 / O_DIM)
      outr[...] = y * lax.rsqrt(var + EPS) * rwr[...]
  return pl.pallas_call(
      body,
      grid=(NPAD // bm, ncs_in),
      in_specs=[
          pl.BlockSpec((NC, 1, bm, FC), lambda i, k: (0, k, i, 0)),
          pl.BlockSpec((1, bm, FC), lambda i, k: (k, i, 0)),
          pl.BlockSpec((1, FC, fo), lambda i, k: (k, 0, 0)),
          pl.BlockSpec((1, fo), lambda i, k: (0, 0)),
          pl.BlockSpec((bm, 1), lambda i, k: (i, 0)),
          pl.BlockSpec((1, fo), lambda i, k: (0, 0)),
      ],
      out_specs=pl.BlockSpec((bm, fo), lambda i, k: (i, 0)),
      out_shape=jax.ShapeDtypeStruct((NPAD, fo), jnp.float32),
      scratch_shapes=[pltpu.VMEM((bm, fo), jnp.float32)],
      compiler_params=pltpu.CompilerParams(
          dimension_semantics=("parallel", "arbitrary")),
  )(zp, xn, wr, b2, dinv, rmsw)


def _pool_cls(hf, batch2, wc, bc2):
  bm = 512
  fo = 2048
  def body(hfr, btr, wcr, bcr, outr, poolr, cntr):
    i = pl.program_id(0)
    ni = pl.num_programs(0)
    @pl.when(i == 0)
    def _():
      poolr[...] = jnp.zeros_like(poolr)
      cntr[...] = jnp.zeros_like(cntr)
    m = (btr[...] == lax.broadcasted_iota(jnp.int32, (bm, G), 1)
         ).astype(jnp.float32)
    poolr[...] += lax.dot_general(m, hfr[...], (((0,), (0,)), ((), ())),
                                  preferred_element_type=jnp.float32)
    cntr[...] += lax.dot_general(m, jnp.ones((bm, 1), jnp.float32),
                                 (((0,), (0,)), ((), ())),
                                 preferred_element_type=jnp.float32)
    @pl.when(i == ni - 1)
    def _():
      pooled = poolr[...] / jnp.maximum(cntr[...], 1.0)
      outr[...] = jnp.dot(pooled, wcr[...],
                          preferred_element_type=jnp.float32) + bcr[...]
  return pl.pallas_call(
      body,
      grid=(NPAD // bm,),
      in_specs=[
          pl.BlockSpec((bm, fo), lambda i: (i, 0)),
          pl.BlockSpec((bm, 1), lambda i: (i, 0)),
          pl.BlockSpec((fo, fo), lambda i: (0, 0)),
          pl.BlockSpec((1, fo), lambda i: (0, 0)),
      ],
      out_specs=pl.BlockSpec((G, fo), lambda i: (0, 0)),
      out_shape=jax.ShapeDtypeStruct((G, fo), jnp.float32),
      scratch_shapes=[pltpu.VMEM((G, fo), jnp.float32),
                      pltpu.VMEM((G, 1), jnp.float32)],
      compiler_params=pltpu.CompilerParams(
          dimension_semantics=("arbitrary",)),
  )(hf, batch2, wc, bc2)


def kernel(hidden_states, edge_index, batch, W1, b1, W2, b2, W3, b3,
           rms_weight, Wc, bc):
  f32 = jnp.float32
  x_p = jnp.pad(hidden_states.astype(f32), ((0, NPAD - N), (0, 0)))
  src = edge_index[0]
  dst = edge_index[1]
  pe = NW * EPT - E
  pad_src = jnp.arange(pe, dtype=jnp.int32) % N
  pad_dst = N + jnp.arange(pe, dtype=jnp.int32) % (NPAD - N)
  srcp = jnp.concatenate([src, pad_src])
  dstp = jnp.concatenate([dst, pad_dst])
  dstp2 = dstp.reshape(NW, NBLK, BI)
  batch2 = jnp.pad(batch, (0, NPAD - N),
                   constant_values=G).reshape(NPAD, 1)

  W1r = W1.reshape(1, FC, H)
  W2r = W2.reshape(H // FC, FC, I_DIM)
  W3r = jnp.pad(W3, ((0, 0), (0, 2048 - O_DIM))).reshape(I_DIM // FC, FC, 2048)
  b1r = b1.reshape(1, H)
  b2r = b2.reshape(1, I_DIM)
  b3r = jnp.pad(b3, (0, 2048 - O_DIM)).reshape(1, 2048)
  rmswr = jnp.pad(rms_weight, (0, 2048 - O_DIM)).reshape(1, 2048)
  wcp = jnp.pad(Wc, ((0, 2048 - O_DIM), (0, 2048 - C_DIM)))
  bcp = jnp.pad(bc, (0, 2048 - C_DIM)).reshape(1, 2048)

  degp = _DEG(dstp2).reshape(NC, NPAD, 1)
  xn0, dinv = _xn0(x_p, degp)
  z1 = _SPMM[1](xn0, srcp, dstp)
  xn1 = _mm_silu(z1, xn0, W1r, b1r, dinv, 1, H)
  z2 = _SPMM[4](xn1, srcp, dstp)
  xn2 = _mm_silu(z2, xn1, W2r, b2r, dinv, 4, I_DIM)
  z3 = _SPMM[16](xn2, srcp, dstp)
  hf = _mm_rms(z3, xn2, W3r, b3r, dinv, rmswr)
  outp = _pool_cls(hf, batch2, wcp, bcp)
  return outp[:, :C_DIM]


# speedup vs baseline: 10.4788x; 1.1526x over previous
import functools

import jax
import jax.numpy as jnp
from jax import lax
from jax.experimental import pallas as pl
from jax.experimental.pallas import tpu as pltpu
from jax.experimental.pallas import tpu_sc as plsc

N = 10000
E = 320000
G = 16
D_IN = 128
H = 512
I_DIM = 2048
O_DIM = 2000
C_DIM = 2000
EPS = 1e-08

NPAD = 10240
FC = 128
NC = 2
NS = 16
NW = NC * NS
EPT = NPAD
BI = 128
NBLK = EPT // BI
BG = 80
NBG = EPT // BG
SBB = 8
NSB = NBG // SBB
IDXW = BG * SBB
RPT = NPAD // NS

_SC_MESH = plsc.VectorSubcoreMesh(
    core_axis_name="c", subcore_axis_name="s", num_cores=NC, num_subcores=NS)


def _make_deg():
  @functools.partial(
      pl.kernel,
      out_type=jax.ShapeDtypeStruct((NC * NPAD,), jnp.float32),
      mesh=_SC_MESH,
      scratch_types=[
          pltpu.VMEM((NBLK, BI), jnp.int32),
          pltpu.VMEM((RPT,), jnp.float32),
          pltpu.VMEM((BI,), jnp.float32),
          pltpu.VMEM_SHARED((NPAD,), jnp.float32),
          pltpu.SemaphoreType.DMA,
      ],
  )
  def deg_kernel(dst_hbm, out_hbm, didx, zbuf, ones_v, acc, sem):
    c = lax.axis_index("c")
    s = lax.axis_index("s")
    wid = c * NS + s
    pltpu.sync_copy(dst_hbm.at[wid], didx)
    zv = jnp.zeros((16,), jnp.float32)
    ov = jnp.ones((16,), jnp.float32)
    def fill(i, _):
      zbuf[pl.ds(i * 16, 16)] = zv
      return 0
    lax.fori_loop(0, RPT // 16, fill, 0)
    for i in range(BI // 16):
      ones_v[pl.ds(i * 16, 16)] = ov
    pltpu.sync_copy(zbuf, acc.at[pl.ds(s * RPT, RPT)])
    plsc.subcore_barrier()

    def round_body(r, _):
      for b in range(8):
        j = r * 8 + b
        pltpu.async_copy(ones_v, acc.at[didx.at[j]], sem, add=True)
      pltpu.make_async_copy(
          dst_hbm.at[0, pl.ds(0, 8)], didx.at[pl.ds(0, 8)], sem).wait()
      return 0
    lax.fori_loop(0, NBLK // 8, round_body, 0)
    plsc.subcore_barrier()
    pltpu.sync_copy(acc.at[pl.ds(s * RPT, RPT)],
                    out_hbm.at[pl.ds(c * NPAD + s * RPT, RPT)])

  return deg_kernel


def _make_spmm(ncs):
  @functools.partial(
      pl.kernel,
      out_type=jax.ShapeDtypeStruct((NC, ncs, NPAD, FC), jnp.float32),
      mesh=_SC_MESH,
      scratch_types=[
          pltpu.VMEM((IDXW,), jnp.int32),
          pltpu.VMEM((IDXW,), jnp.int32),
          pltpu.VMEM((IDXW,), jnp.int32),
          pltpu.VMEM((IDXW,), jnp.int32),
          pltpu.VMEM((4, BG, FC), jnp.float32),
          pltpu.VMEM_SHARED((NPAD, FC), jnp.float32),
          [pltpu.SemaphoreType.DMA] * 4,
          [pltpu.SemaphoreType.DMA] * 4,
          [pltpu.SemaphoreType.DMA] * 2,
      ],
  )
  def spmm_kernel(xn_hbm, src_hbm, dst_hbm, out_hbm,
                  sbuf0, sbuf1, dbuf0, dbuf1, rows, acc, gsem, ssem, isem):
    sbufs = (sbuf0, sbuf1)
    dbufs = (dbuf0, dbuf1)
    c = lax.axis_index("c")
    s = lax.axis_index("s")
    wid = c * NS + s
    zv = jnp.zeros((16,), jnp.float32)

    def gwait(b):
      pltpu.make_async_copy(
          xn_hbm.at[0, pl.ds(0, BG)], rows.at[b], gsem[b]).wait()

    def swait(b):
      pltpu.make_async_copy(
          xn_hbm.at[0, pl.ds(0, BG)], rows.at[b], ssem[b]).wait()

    def iwait(ib):
      pltpu.make_async_copy(
          src_hbm.at[pl.ds(0, IDXW)], sbufs[ib], isem[ib]).wait()
      pltpu.make_async_copy(
          src_hbm.at[pl.ds(0, IDXW)], dbufs[ib], isem[ib]).wait()

    def refill(ib, sb_next):
      off = wid * EPT + sb_next * IDXW
      pltpu.async_copy(src_hbm.at[pl.ds(off, IDXW)], sbufs[ib], isem[ib])
      pltpu.async_copy(dst_hbm.at[pl.ds(off, IDXW)], dbufs[ib], isem[ib])

    def gissue(ci, ib, jl, b):
      pltpu.async_copy(xn_hbm.at[ci].at[sbufs[ib].at[pl.ds(jl * BG, BG)]],
                       rows.at[b], gsem[b])

    for ci in range(ncs):
      def fillz(i, _):
        for k in range(FC // 16):
          rows[0, i, pl.ds(k * 16, 16)] = zv
        return 0
      lax.fori_loop(0, BG, fillz, 0)
      for z in range(RPT // BG):
        pltpu.sync_copy(rows.at[0], acc.at[pl.ds(s * RPT + z * BG, BG)])
      plsc.subcore_barrier()

      pltpu.sync_copy(src_hbm.at[pl.ds(wid * EPT, IDXW)], sbuf0)
      pltpu.sync_copy(dst_hbm.at[pl.ds(wid * EPT, IDXW)], dbuf0)
      gissue(ci, 0, 0, 0)
      gissue(ci, 0, 1, 1)

      def pair(p, _):
        for sbp in range(2):
          ib = sbp
          io = 1 - sbp
          sb = p * 2 + sbp
          for jl in range(SBB):
            b = jl % 4
            br = (jl + 2) % 4
            gwait(b)
            pltpu.async_copy(
                rows.at[b], acc.at[dbufs[ib].at[pl.ds(jl * BG, BG)]],
                ssem[b], add=True)
            if jl == 3:
              if sbp == 0:
                refill(io, sb + 1)
              else:
                @pl.when(p < NSB // 2 - 1)
                def _():
                  refill(io, sb + 1)
            if sbp == 0 and jl < 2:
              @pl.when(p > 0)
              def _():
                swait(br)
              gissue(ci, ib, jl + 2, br)
            elif jl < SBB - 2:
              swait(br)
              gissue(ci, ib, jl + 2, br)
            else:
              if sbp == 0:
                swait(br)
                if jl == SBB - 2:
                  iwait(io)
                gissue(ci, io, jl - (SBB - 2), br)
              else:
                @pl.when(p < NSB // 2 - 1)
                def _():
                  swait(br)
                  if jl == SBB - 2:
                    iwait(io)
                  gissue(ci, io, jl - (SBB - 2), br)
        return 0
      lax.fori_loop(0, NSB // 2, pair, 0)
      for b in range(4):
        swait(b)
      plsc.subcore_barrier()
      pltpu.sync_copy(acc.at[pl.ds(s * RPT, RPT)],
                      out_hbm.at[c, ci, pl.ds(s * RPT, RPT)])

  return spmm_kernel


_DEG = _make_deg()
_SPMM = {ncs: _make_spmm(ncs) for ncs in (1, 4, 16)}


def _xn0(x_p, degp):
  bm = 512
  def body(xr, dgr, xnr, dvr):
    d = sum(dgr[cc] for cc in range(NC)) + 1.0
    dv = lax.rsqrt(d)
    dvr[...] = dv
    xnr[0] = xr[...] * dv
  return pl.pallas_call(
      body,
      grid=(NPAD // bm,),
      in_specs=[
          pl.BlockSpec((bm, FC), lambda i: (i, 0)),
          pl.BlockSpec((NC, bm, 1), lambda i: (0, i, 0)),
      ],
      out_specs=(
          pl.BlockSpec((1, bm, FC), lambda i: (0, i, 0)),
          pl.BlockSpec((bm, 1), lambda i: (i, 0)),
      ),
      out_shape=(
          jax.ShapeDtypeStruct((1, NPAD, FC), jnp.float32),
          jax.ShapeDtypeStruct((NPAD, 1), jnp.float32),
      ),
  )(x_p, degp)


def _mm_silu(zp, xn, wr, b2, dinv, ncs_in, fout):
  bm = 1024
  bn = 256
  ncs_out = fout // FC
  def body(zpr, xnr, wrr, brr, dvr, outr, accr):
    k = pl.program_id(2)
    @pl.when(k == 0)
    def _():
      accr[...] = jnp.zeros_like(accr)
    z = sum(zpr[cc, 0] for cc in range(NC)) + xnr[0]
    accr[...] += jnp.dot(z.astype(jnp.bfloat16), wrr[0],
                         preferred_element_type=jnp.float32)
    @pl.when(k == ncs_in - 1)
    def _():
      dv = dvr[...]
      y = accr[...] * dv + brr[...]
      h = y * jax.nn.sigmoid(y)
      o = h * dv
      for t in range(bn // FC):
        outr[t] = o[:, t * FC:(t + 1) * FC]
  return pl.pallas_call(
      body,
      grid=(NPAD // bm, fout // bn, ncs_in),
      in_specs=[
          pl.BlockSpec((NC, 1, bm, FC), lambda i, j, k: (0, k, i, 0)),
          pl.BlockSpec((1, bm, FC), lambda i, j, k: (k, i, 0)),
          pl.BlockSpec((1, FC, bn), lambda i, j, k: (k, 0, j)),
          pl.BlockSpec((1, bn), lambda i, j, k: (0, j)),
          pl.BlockSpec((bm, 1), lambda i, j, k: (i, 0)),
      ],
      out_specs=pl.BlockSpec((bn // FC, bm, FC), lambda i, j, k: (j, i, 0)),
      out_shape=jax.ShapeDtypeStruct((ncs_out, NPAD, FC), jnp.float32),
      scratch_shapes=[pltpu.VMEM((bm, bn), jnp.float32)],
      compiler_params=pltpu.CompilerParams(
          dimension_semantics=("parallel", "parallel", "arbitrary")),
  )(zp, xn, wr, b2, dinv)


def _mm_rms(zp, xn, wr, b2, dinv, rmsw):
  bm = 512
  ncs_in = I_DIM // FC
  fo = 2048
  def body(zpr, xnr, wrr, brr, dvr, rwr, outr, accr):
    k = pl.program_id(1)
    @pl.when(k == 0)
    def _():
      accr[...] = jnp.zeros_like(accr)
    z = sum(zpr[cc, 0] for cc in range(NC)) + xnr[0]
    accr[...] += jnp.dot(z.astype(jnp.bfloat16), wrr[0],
                         preferred_element_type=jnp.float32)
    @pl.when(k == ncs_in - 1)
    def _():
      y = accr[...] * dvr[...] + brr[...]
      var = jnp.sum(y * y, axis=1, keepdims=True) * (1.0 / O_DIM)
      outr[...] = y * lax.rsqrt(var + EPS) * rwr[...]
  return pl.pallas_call(
      body,
      grid=(NPAD // bm, ncs_in),
      in_specs=[
          pl.BlockSpec((NC, 1, bm, FC), lambda i, k: (0, k, i, 0)),
          pl.BlockSpec((1, bm, FC), lambda i, k: (k, i, 0)),
          pl.BlockSpec((1, FC, fo), lambda i, k: (k, 0, 0)),
          pl.BlockSpec((1, fo), lambda i, k: (0, 0)),
          pl.BlockSpec((bm, 1), lambda i, k: (i, 0)),
          pl.BlockSpec((1, fo), lambda i, k: (0, 0)),
      ],
      out_specs=pl.BlockSpec((bm, fo), lambda i, k: (i, 0)),
      out_shape=jax.ShapeDtypeStruct((NPAD, fo), jnp.float32),
      scratch_shapes=[pltpu.VMEM((bm, fo), jnp.float32)],
      compiler_params=pltpu.CompilerParams(
          dimension_semantics=("parallel", "arbitrary")),
  )(zp, xn, wr, b2, dinv, rmsw)


def _pool_cls(hf, batch2, wc, bc2):
  bm = 512
  fo = 2048
  def body(hfr, btr, wcr, bcr, outr, poolr, cntr):
    i = pl.program_id(0)
    ni = pl.num_programs(0)
    @pl.when(i == 0)
    def _():
      poolr[...] = jnp.zeros_like(poolr)
      cntr[...] = jnp.zeros_like(cntr)
    m = (btr[...] == lax.broadcasted_iota(jnp.int32, (bm, G), 1)
         ).astype(jnp.float32)
    poolr[...] += lax.dot_general(m, hfr[...], (((0,), (0,)), ((), ())),
                                  preferred_element_type=jnp.float32)
    cntr[...] += lax.dot_general(m, jnp.ones((bm, 1), jnp.float32),
                                 (((0,), (0,)), ((), ())),
                                 preferred_element_type=jnp.float32)
    @pl.when(i == ni - 1)
    def _():
      pooled = poolr[...] / jnp.maximum(cntr[...], 1.0)
      outr[...] = jnp.dot(pooled, wcr[...],
                          preferred_element_type=jnp.float32) + bcr[...]
  return pl.pallas_call(
      body,
      grid=(NPAD // bm,),
      in_specs=[
          pl.BlockSpec((bm, fo), lambda i: (i, 0)),
          pl.BlockSpec((bm, 1), lambda i: (i, 0)),
          pl.BlockSpec((fo, fo), lambda i: (0, 0)),
          pl.BlockSpec((1, fo), lambda i: (0, 0)),
      ],
      out_specs=pl.BlockSpec((G, fo), lambda i: (0, 0)),
      out_shape=jax.ShapeDtypeStruct((G, fo), jnp.float32),
      scratch_shapes=[pltpu.VMEM((G, fo), jnp.float32),
                      pltpu.VMEM((G, 1), jnp.float32)],
      compiler_params=pltpu.CompilerParams(
          dimension_semantics=("arbitrary",)),
  )(hf, batch2, wc, bc2)


def kernel(hidden_states, edge_index, batch, W1, b1, W2, b2, W3, b3,
           rms_weight, Wc, bc):
  f32 = jnp.float32
  x_p = jnp.pad(hidden_states.astype(f32), ((0, NPAD - N), (0, 0)))
  src = edge_index[0]
  dst = edge_index[1]
  pe = NW * EPT - E
  pad_src = jnp.arange(pe, dtype=jnp.int32) % N
  pad_dst = N + jnp.arange(pe, dtype=jnp.int32) % (NPAD - N)
  srcp = jnp.concatenate([src, pad_src])
  dstp = jnp.concatenate([dst, pad_dst])
  dstp2 = dstp.reshape(NW, NBLK, BI)
  batch2 = jnp.pad(batch, (0, NPAD - N),
                   constant_values=G).reshape(NPAD, 1)

  bf16 = jnp.bfloat16
  W1r = W1.reshape(1, FC, H).astype(bf16)
  W2r = W2.reshape(H // FC, FC, I_DIM).astype(bf16)
  W3r = jnp.pad(W3, ((0, 0), (0, 2048 - O_DIM))).reshape(
      I_DIM // FC, FC, 2048).astype(bf16)
  b1r = b1.reshape(1, H)
  b2r = b2.reshape(1, I_DIM)
  b3r = jnp.pad(b3, (0, 2048 - O_DIM)).reshape(1, 2048)
  rmswr = jnp.pad(rms_weight, (0, 2048 - O_DIM)).reshape(1, 2048)
  wcp = jnp.pad(Wc, ((0, 2048 - O_DIM), (0, 2048 - C_DIM)))
  bcp = jnp.pad(bc, (0, 2048 - C_DIM)).reshape(1, 2048)

  degp = _DEG(dstp2).reshape(NC, NPAD, 1)
  xn0, dinv = _xn0(x_p, degp)
  z1 = _SPMM[1](xn0, srcp, dstp)
  xn1 = _mm_silu(z1, xn0, W1r, b1r, dinv, 1, H)
  z2 = _SPMM[4](xn1, srcp, dstp)
  xn2 = _mm_silu(z2, xn1, W2r, b2r, dinv, 4, I_DIM)
  z3 = _SPMM[16](xn2, srcp, dstp)
  hf = _mm_rms(z3, xn2, W3r, b3r, dinv, rmswr)
  outp = _pool_cls(hf, batch2, wcp, bcp)
  return outp[:, :C_DIM]


# speedup vs baseline: 10.6450x; 1.0159x over previous
import functools

import jax
import jax.numpy as jnp
from jax import lax
from jax.experimental import pallas as pl
from jax.experimental.pallas import tpu as pltpu
from jax.experimental.pallas import tpu_sc as plsc

N = 10000
E = 320000
G = 16
D_IN = 128
H = 512
I_DIM = 2048
O_DIM = 2000
C_DIM = 2000
EPS = 1e-08

NPAD = 10240
FC = 128
NC = 2
NS = 16
NW = NC * NS
EPT = NPAD
BI = 128
NBLK = EPT // BI
BG = 80
NBG = EPT // BG
SBB = 8
NSB = NBG // SBB
IDXW = BG * SBB
RPT = NPAD // NS

_SC_MESH = plsc.VectorSubcoreMesh(
    core_axis_name="c", subcore_axis_name="s", num_cores=NC, num_subcores=NS)


def _make_deg():
  @functools.partial(
      pl.kernel,
      out_type=jax.ShapeDtypeStruct((NC * NPAD,), jnp.float32),
      mesh=_SC_MESH,
      scratch_types=[
          pltpu.VMEM((NBLK, BI), jnp.int32),
          pltpu.VMEM((RPT,), jnp.float32),
          pltpu.VMEM((BI,), jnp.float32),
          pltpu.VMEM_SHARED((NPAD,), jnp.float32),
          pltpu.SemaphoreType.DMA,
      ],
  )
  def deg_kernel(dst_hbm, out_hbm, didx, zbuf, ones_v, acc, sem):
    c = lax.axis_index("c")
    s = lax.axis_index("s")
    wid = c * NS + s
    pltpu.sync_copy(dst_hbm.at[wid], didx)
    zv = jnp.zeros((16,), jnp.float32)
    ov = jnp.ones((16,), jnp.float32)
    def fill(i, _):
      zbuf[pl.ds(i * 16, 16)] = zv
      return 0
    lax.fori_loop(0, RPT // 16, fill, 0)
    for i in range(BI // 16):
      ones_v[pl.ds(i * 16, 16)] = ov
    pltpu.sync_copy(zbuf, acc.at[pl.ds(s * RPT, RPT)])
    plsc.subcore_barrier()

    def round_body(r, _):
      for b in range(8):
        j = r * 8 + b
        pltpu.async_copy(ones_v, acc.at[didx.at[j]], sem, add=True)
      pltpu.make_async_copy(
          dst_hbm.at[0, pl.ds(0, 8)], didx.at[pl.ds(0, 8)], sem).wait()
      return 0
    lax.fori_loop(0, NBLK // 8, round_body, 0)
    plsc.subcore_barrier()
    pltpu.sync_copy(acc.at[pl.ds(s * RPT, RPT)],
                    out_hbm.at[pl.ds(c * NPAD + s * RPT, RPT)])

  return deg_kernel


def _make_spmm(ncs):
  @functools.partial(
      pl.kernel,
      out_type=jax.ShapeDtypeStruct((NC, ncs, NPAD, FC), jnp.float32),
      mesh=_SC_MESH,
      scratch_types=[
          pltpu.VMEM((IDXW,), jnp.int32),
          pltpu.VMEM((IDXW,), jnp.int32),
          pltpu.VMEM((IDXW,), jnp.int32),
          pltpu.VMEM((IDXW,), jnp.int32),
          pltpu.VMEM((4, BG, FC), jnp.float32),
          pltpu.VMEM_SHARED((NPAD, FC), jnp.float32),
          [pltpu.SemaphoreType.DMA] * 4,
          [pltpu.SemaphoreType.DMA] * 4,
          [pltpu.SemaphoreType.DMA] * 2,
      ],
  )
  def spmm_kernel(xn_hbm, src_hbm, dst_hbm, out_hbm,
                  sbuf0, sbuf1, dbuf0, dbuf1, rows, acc, gsem, ssem, isem):
    sbufs = (sbuf0, sbuf1)
    dbufs = (dbuf0, dbuf1)
    c = lax.axis_index("c")
    s = lax.axis_index("s")
    wid = c * NS + s
    zv = jnp.zeros((16,), jnp.float32)

    def gwait(b):
      pltpu.make_async_copy(
          xn_hbm.at[0, pl.ds(0, BG)], rows.at[b], gsem[b]).wait()

    def swait(b):
      pltpu.make_async_copy(
          xn_hbm.at[0, pl.ds(0, BG)], rows.at[b], ssem[b]).wait()

    def iwait(ib):
      pltpu.make_async_copy(
          src_hbm.at[pl.ds(0, IDXW)], sbufs[ib], isem[ib]).wait()
      pltpu.make_async_copy(
          src_hbm.at[pl.ds(0, IDXW)], dbufs[ib], isem[ib]).wait()

    def refill(ib, sb_next):
      off = wid * EPT + sb_next * IDXW
      pltpu.async_copy(src_hbm.at[pl.ds(off, IDXW)], sbufs[ib], isem[ib])
      pltpu.async_copy(dst_hbm.at[pl.ds(off, IDXW)], dbufs[ib], isem[ib])

    def gissue(ci, ib, jl, b):
      pltpu.async_copy(xn_hbm.at[ci].at[sbufs[ib].at[pl.ds(jl * BG, BG)]],
                       rows.at[b], gsem[b])

    for ci in range(ncs):
      def fillz(i, _):
        for k in range(FC // 16):
          rows[0, i, pl.ds(k * 16, 16)] = zv
        return 0
      lax.fori_loop(0, BG, fillz, 0)
      for z in range(RPT // BG):
        pltpu.sync_copy(rows.at[0], acc.at[pl.ds(s * RPT + z * BG, BG)])
      plsc.subcore_barrier()

      pltpu.sync_copy(src_hbm.at[pl.ds(wid * EPT, IDXW)], sbuf0)
      pltpu.sync_copy(dst_hbm.at[pl.ds(wid * EPT, IDXW)], dbuf0)
      gissue(ci, 0, 0, 0)
      gissue(ci, 0, 1, 1)

      def pair(p, _):
        for sbp in range(2):
          ib = sbp
          io = 1 - sbp
          sb = p * 2 + sbp
          for jl in range(SBB):
            b = jl % 4
            br = (jl + 2) % 4
            gwait(b)
            pltpu.async_copy(
                rows.at[b], acc.at[dbufs[ib].at[pl.ds(jl * BG, BG)]],
                ssem[b], add=True)
            if jl == 3:
              if sbp == 0:
                refill(io, sb + 1)
              else:
                @pl.when(p < NSB // 2 - 1)
                def _():
                  refill(io, sb + 1)
            if sbp == 0 and jl < 2:
              @pl.when(p > 0)
              def _():
                swait(br)
              gissue(ci, ib, jl + 2, br)
            elif jl < SBB - 2:
              swait(br)
              gissue(ci, ib, jl + 2, br)
            else:
              if sbp == 0:
                swait(br)
                if jl == SBB - 2:
                  iwait(io)
                gissue(ci, io, jl - (SBB - 2), br)
              else:
                @pl.when(p < NSB // 2 - 1)
                def _():
                  swait(br)
                  if jl == SBB - 2:
                    iwait(io)
                  gissue(ci, io, jl - (SBB - 2), br)
        return 0
      lax.fori_loop(0, NSB // 2, pair, 0)
      for b in range(4):
        swait(b)
      plsc.subcore_barrier()
      pltpu.sync_copy(acc.at[pl.ds(s * RPT, RPT)],
                      out_hbm.at[c, ci, pl.ds(s * RPT, RPT)])

  return spmm_kernel


_DEG = _make_deg()
_SPMM = {ncs: _make_spmm(ncs) for ncs in (1, 4, 16)}


def _xn0(x_p, degp):
  bm = 512
  def body(xr, dgr, xnr, dvr):
    d = sum(dgr[cc] for cc in range(NC)) + 1.0
    dv = lax.rsqrt(d)
    dvr[...] = dv
    xnr[0] = xr[...] * dv
  return pl.pallas_call(
      body,
      grid=(NPAD // bm,),
      in_specs=[
          pl.BlockSpec((bm, FC), lambda i: (i, 0)),
          pl.BlockSpec((NC, bm, 1), lambda i: (0, i, 0)),
      ],
      out_specs=(
          pl.BlockSpec((1, bm, FC), lambda i: (0, i, 0)),
          pl.BlockSpec((bm, 1), lambda i: (i, 0)),
      ),
      out_shape=(
          jax.ShapeDtypeStruct((1, NPAD, FC), jnp.float32),
          jax.ShapeDtypeStruct((NPAD, 1), jnp.float32),
      ),
  )(x_p, degp)


def _mm_silu(zp, xn, wr, b2, dinv, ncs_in, fout):
  bm = 1024
  bn = 256
  ncs_out = fout // FC
  def body(zpr, xnr, wrr, brr, dvr, outr, accr):
    k = pl.program_id(2)
    @pl.when(k == 0)
    def _():
      accr[...] = jnp.zeros_like(accr)
    z = sum(zpr[cc, 0] for cc in range(NC)) + xnr[0]
    accr[...] += jnp.dot(z.astype(jnp.bfloat16), wrr[0],
                         preferred_element_type=jnp.float32)
    @pl.when(k == ncs_in - 1)
    def _():
      dv = dvr[...]
      y = accr[...] * dv + brr[...]
      h = y * jax.nn.sigmoid(y)
      o = h * dv
      for t in range(bn // FC):
        outr[t] = o[:, t * FC:(t + 1) * FC]
  return pl.pallas_call(
      body,
      grid=(NPAD // bm, fout // bn, ncs_in),
      in_specs=[
          pl.BlockSpec((NC, 1, bm, FC), lambda i, j, k: (0, k, i, 0)),
          pl.BlockSpec((1, bm, FC), lambda i, j, k: (k, i, 0)),
          pl.BlockSpec((1, FC, bn), lambda i, j, k: (k, 0, j)),
          pl.BlockSpec((1, bn), lambda i, j, k: (0, j)),
          pl.BlockSpec((bm, 1), lambda i, j, k: (i, 0)),
      ],
      out_specs=pl.BlockSpec((bn // FC, bm, FC), lambda i, j, k: (j, i, 0)),
      out_shape=jax.ShapeDtypeStruct((ncs_out, NPAD, FC), jnp.float32),
      scratch_shapes=[pltpu.VMEM((bm, bn), jnp.float32)],
      compiler_params=pltpu.CompilerParams(
          dimension_semantics=("parallel", "parallel", "arbitrary")),
  )(zp, xn, wr, b2, dinv)


def _mm_rms_pool_cls(zp, xn, wr, b2, dinv, rmsw, batch2, wc, bc2):
  bm = 512
  ncs_in = I_DIM // FC
  fo = 2048
  def body(zpr, xnr, wrr, brr, dvr, rwr, btr, wcr, bcr, outr, accr,
           poolr, cntr):
    i = pl.program_id(0)
    ni = pl.num_programs(0)
    k = pl.program_id(1)
    @pl.when(k == 0)
    def _():
      accr[...] = jnp.zeros_like(accr)
    @pl.when(jnp.logical_and(i == 0, k == 0))
    def _():
      poolr[...] = jnp.zeros_like(poolr)
      cntr[...] = jnp.zeros_like(cntr)
    z = sum(zpr[cc, 0] for cc in range(NC)) + xnr[0]
    accr[...] += jnp.dot(z.astype(jnp.bfloat16), wrr[0],
                         preferred_element_type=jnp.float32)
    @pl.when(k == ncs_in - 1)
    def _():
      y = accr[...] * dvr[...] + brr[...]
      var = jnp.sum(y * y, axis=1, keepdims=True) * (1.0 / O_DIM)
      hf = y * lax.rsqrt(var + EPS) * rwr[...]
      m = (btr[...] == lax.broadcasted_iota(jnp.int32, (bm, G), 1)
           ).astype(jnp.float32)
      poolr[...] += lax.dot_general(m, hf, (((0,), (0,)), ((), ())),
                                    preferred_element_type=jnp.float32)
      cntr[...] += lax.dot_general(m, jnp.ones((bm, 1), jnp.float32),
                                   (((0,), (0,)), ((), ())),
                                   preferred_element_type=jnp.float32)
      @pl.when(i == ni - 1)
      def _():
        pooled = poolr[...] / jnp.maximum(cntr[...], 1.0)
        outr[...] = jnp.dot(pooled, wcr[...],
                            preferred_element_type=jnp.float32) + bcr[...]
  return pl.pallas_call(
      body,
      grid=(NPAD // bm, ncs_in),
      in_specs=[
          pl.BlockSpec((NC, 1, bm, FC), lambda i, k: (0, k, i, 0)),
          pl.BlockSpec((1, bm, FC), lambda i, k: (k, i, 0)),
          pl.BlockSpec((1, FC, fo), lambda i, k: (k, 0, 0)),
          pl.BlockSpec((1, fo), lambda i, k: (0, 0)),
          pl.BlockSpec((bm, 1), lambda i, k: (i, 0)),
          pl.BlockSpec((1, fo), lambda i, k: (0, 0)),
          pl.BlockSpec((bm, 1), lambda i, k: (i, 0)),
          pl.BlockSpec((fo, fo), lambda i, k: (0, 0)),
          pl.BlockSpec((1, fo), lambda i, k: (0, 0)),
      ],
      out_specs=pl.BlockSpec((G, fo), lambda i, k: (0, 0)),
      out_shape=jax.ShapeDtypeStruct((G, fo), jnp.float32),
      scratch_shapes=[pltpu.VMEM((bm, fo), jnp.float32),
                      pltpu.VMEM((G, fo), jnp.float32),
                      pltpu.VMEM((G, 1), jnp.float32)],
      compiler_params=pltpu.CompilerParams(
          dimension_semantics=("arbitrary", "arbitrary")),
  )(zp, xn, wr, b2, dinv, rmsw, batch2, wc, bc2)


def kernel(hidden_states, edge_index, batch, W1, b1, W2, b2, W3, b3,
           rms_weight, Wc, bc):
  f32 = jnp.float32
  x_p = jnp.pad(hidden_states.astype(f32), ((0, NPAD - N), (0, 0)))
  src = edge_index[0]
  dst = edge_index[1]
  pe = NW * EPT - E
  pad_src = jnp.arange(pe, dtype=jnp.int32) % N
  pad_dst = N + jnp.arange(pe, dtype=jnp.int32) % (NPAD - N)
  srcp = jnp.concatenate([src, pad_src])
  dstp = jnp.concatenate([dst, pad_dst])
  dstp2 = dstp.reshape(NW, NBLK, BI)
  batch2 = jnp.pad(batch, (0, NPAD - N),
                   constant_values=G).reshape(NPAD, 1)

  bf16 = jnp.bfloat16
  W1r = W1.reshape(1, FC, H).astype(bf16)
  W2r = W2.reshape(H // FC, FC, I_DIM).astype(bf16)
  W3r = jnp.pad(W3, ((0, 0), (0, 2048 - O_DIM))).reshape(
      I_DIM // FC, FC, 2048).astype(bf16)
  b1r = b1.reshape(1, H)
  b2r = b2.reshape(1, I_DIM)
  b3r = jnp.pad(b3, (0, 2048 - O_DIM)).reshape(1, 2048)
  rmswr = jnp.pad(rms_weight, (0, 2048 - O_DIM)).reshape(1, 2048)
  wcp = jnp.pad(Wc, ((0, 2048 - O_DIM), (0, 2048 - C_DIM)))
  bcp = jnp.pad(bc, (0, 2048 - C_DIM)).reshape(1, 2048)

  degp = _DEG(dstp2).reshape(NC, NPAD, 1)
  xn0, dinv = _xn0(x_p, degp)
  z1 = _SPMM[1](xn0, srcp, dstp)
  xn1 = _mm_silu(z1, xn0, W1r, b1r, dinv, 1, H)
  z2 = _SPMM[4](xn1, srcp, dstp)
  xn2 = _mm_silu(z2, xn1, W2r, b2r, dinv, 4, I_DIM)
  z3 = _SPMM[16](xn2, srcp, dstp)
  outp = _mm_rms_pool_cls(z3, xn2, W3r, b3r, dinv, rmswr, batch2, wcp, bcp)
  return outp[:, :C_DIM]
